# baseline (speedup 1.0000x reference)
import dataclasses
import functools

import jax
import jax.numpy as jnp
from jax.experimental import pallas as pl
from jax.experimental.pallas import tpu as pltpu
from jax.experimental.pallas import tpu_sc as plsc


def _sc_gather(emb, idx):
    s = idx.shape[0]
    e = emb.shape[1]
    info = plsc.get_sparse_core_info()
    nc, ns = info.num_cores, info.num_subcores
    nw = nc * ns
    assert s % (8 * nw) == 0 and e % info.num_lanes == 0
    b_per_w = s // nw
    mesh = plsc.VectorSubcoreMesh(core_axis_name="c", subcore_axis_name="s")

    @functools.partial(
        pl.kernel,
        out_type=jax.ShapeDtypeStruct((s, e), emb.dtype),
        mesh=mesh,
        scratch_types=[
            pltpu.VMEM((b_per_w,), jnp.int32),
            pltpu.VMEM((b_per_w, e), emb.dtype),
            pltpu.SemaphoreType.DMA,
        ],
    )
    def gather_kernel(emb_hbm, i_hbm, o_hbm, idx_v, rows_v, sem):
        wid = jax.lax.axis_index("s") * nc + jax.lax.axis_index("c")
        base = wid * b_per_w
        pltpu.sync_copy(i_hbm.at[pl.ds(base, b_per_w)], idx_v)
        pltpu.async_copy(emb_hbm.at[idx_v], rows_v, sem).wait()
        pltpu.sync_copy(rows_v, o_hbm.at[pl.ds(base, b_per_w)])

    return gather_kernel(emb, idx)


def _gate(g, h):
    i = jax.nn.sigmoid(g[:, :h])
    gg = jnp.tanh(g[:, h : 2 * h])
    o = jax.nn.sigmoid(g[:, 2 * h :])
    return o * jnp.tanh(i * gg)


def _dense_kernel(xe_ref, w0f_ref, w0b_ref, w1f_ref, w1b_ref,
                  b0f_ref, b0b_ref, b1f_ref, b1b_ref,
                  fca_ref, fcb_ref, fcbias_ref, out_ref, *, hdim, precision):
    xe = xe_ref[...]
    dot = functools.partial(
        jnp.dot, preferred_element_type=jnp.float32, precision=precision
    )
    g0f = dot(xe, w0f_ref[...]) + b0f_ref[...]
    g0b = dot(xe, w0b_ref[...]) + b0b_ref[...]
    h0f = _gate(g0f, hdim)
    h0b = _gate(g0b, hdim)
    h0 = jnp.concatenate([h0f, h0b], axis=1)
    g1f = dot(h0, w1f_ref[...]) + b1f_ref[...]
    g1b = dot(h0, w1b_ref[...]) + b1b_ref[...]
    h1 = jnp.concatenate([_gate(g1f, hdim), _gate(g1b, hdim)], axis=1)
    out = (
        dot(jnp.maximum(xe, 0.0), fca_ref[...])
        + dot(jnp.maximum(h1, 0.0), fcb_ref[...])
        + fcbias_ref[...]
    )
    out_ref[...] = out


def _slice_gates(w):
    h4 = w.shape[0]
    h = h4 // 4
    return jnp.concatenate([w[:h], w[2 * h :]], axis=0)


def _dense(xe, lstm, fcW, fcb, *, block_s=512, precision=jax.lax.Precision.HIGHEST,
           interpret=False):
    s, e = xe.shape
    h = lstm[0][0]["Whh"].shape[1]
    wts = []
    bs = []
    for layer in lstm:
        for d in layer:
            wts.append(_slice_gates(d["Wih"]).T)
            bs.append(_slice_gates((d["bih"] + d["bhh"])[:, None]).T)
    w0f, w0b, w1f, w1b = wts
    b0f, b0b, b1f, b1b = bs
    fcwt = fcW.T
    fca, fcb_w = fcwt[:e], fcwt[e:]
    out_dim = fcW.shape[0]

    grid = (s // block_s,)
    full = lambda arr: pl.BlockSpec(arr.shape, lambda i: (0,) * arr.ndim)
    return pl.pallas_call(
        functools.partial(_dense_kernel, hdim=h, precision=precision),
        grid=grid,
        in_specs=[
            pl.BlockSpec((block_s, e), lambda i: (i, 0)),
            full(w0f), full(w0b), full(w1f), full(w1b),
            full(b0f), full(b0b), full(b1f), full(b1b),
            full(fca), full(fcb_w), full(fcb[None, :]),
        ],
        out_specs=pl.BlockSpec((block_s, out_dim), lambda i: (i, 0)),
        out_shape=jax.ShapeDtypeStruct((s, out_dim), jnp.float32),
        interpret=interpret,
    )(xe, w0f, w0b, w1f, w1b, b0f, b0b, b1f, b1b, fca, fcb_w, fcb[None, :])


def kernel(x, x_len, emb, lstm, fcW, fcb):
    del x_len
    xe = _sc_gather(emb, x.reshape(-1).astype(jnp.int32))
    return _dense(xe, lstm, fcW, fcb)


# speedup vs baseline: 2.4917x; 2.4917x over previous
import dataclasses
import functools

import jax
import jax.numpy as jnp
from jax.experimental import pallas as pl
from jax.experimental.pallas import tpu as pltpu
from jax.experimental.pallas import tpu_sc as plsc


def _sc_gather(emb, idx):
    s = idx.shape[0]
    e = emb.shape[1]
    info = plsc.get_sparse_core_info()
    nc, ns = info.num_cores, info.num_subcores
    nw = nc * ns
    assert s % (8 * nw) == 0 and e % info.num_lanes == 0
    b_per_w = s // nw
    mesh = plsc.VectorSubcoreMesh(core_axis_name="c", subcore_axis_name="s")

    @functools.partial(
        pl.kernel,
        out_type=jax.ShapeDtypeStruct((s, e), emb.dtype),
        mesh=mesh,
        scratch_types=[
            pltpu.VMEM((b_per_w,), jnp.int32),
            pltpu.VMEM((b_per_w, e), emb.dtype),
            pltpu.SemaphoreType.DMA,
        ],
    )
    def gather_kernel(emb_hbm, i_hbm, o_hbm, idx_v, rows_v, sem):
        wid = jax.lax.axis_index("s") * nc + jax.lax.axis_index("c")
        base = wid * b_per_w
        pltpu.sync_copy(i_hbm.at[pl.ds(base, b_per_w)], idx_v)
        pltpu.async_copy(emb_hbm.at[idx_v], rows_v, sem).wait()
        pltpu.sync_copy(rows_v, o_hbm.at[pl.ds(base, b_per_w)])

    return gather_kernel(emb, idx)


def _gate(g, h):
    i = jax.nn.sigmoid(g[:, :h])
    gg = jnp.tanh(g[:, h : 2 * h])
    o = jax.nn.sigmoid(g[:, 2 * h :])
    return o * jnp.tanh(i * gg)


def _dense_kernel(xe_ref, w0f_ref, w0b_ref, w1f_ref, w1b_ref,
                  b0f_ref, b0b_ref, b1f_ref, b1b_ref,
                  fca_ref, fcb_ref, fcbias_ref, out_ref, *, hdim, precision):
    xe = xe_ref[...]
    dot = functools.partial(
        jnp.dot, preferred_element_type=jnp.float32, precision=precision
    )
    g0f = dot(xe, w0f_ref[...]) + b0f_ref[...]
    g0b = dot(xe, w0b_ref[...]) + b0b_ref[...]
    h0f = _gate(g0f, hdim)
    h0b = _gate(g0b, hdim)
    h0 = jnp.concatenate([h0f, h0b], axis=1)
    g1f = dot(h0, w1f_ref[...]) + b1f_ref[...]
    g1b = dot(h0, w1b_ref[...]) + b1b_ref[...]
    h1 = jnp.concatenate([_gate(g1f, hdim), _gate(g1b, hdim)], axis=1)
    out = (
        dot(jnp.maximum(xe, 0.0), fca_ref[...])
        + dot(jnp.maximum(h1, 0.0), fcb_ref[...])
        + fcbias_ref[...]
    )
    out_ref[...] = out


def _slice_gates(w):
    h4 = w.shape[0]
    h = h4 // 4
    return jnp.concatenate([w[:h], w[2 * h :]], axis=0)


def _dense(xe, lstm, fcW, fcb, *, block_s=512, precision=jax.lax.Precision.DEFAULT,
           interpret=False):
    s, e = xe.shape
    h = lstm[0][0]["Whh"].shape[1]
    wts = []
    bs = []
    for layer in lstm:
        for d in layer:
            wts.append(_slice_gates(d["Wih"]).T)
            bs.append(_slice_gates((d["bih"] + d["bhh"])[:, None]).T)
    w0f, w0b, w1f, w1b = wts
    b0f, b0b, b1f, b1b = bs
    fcwt = fcW.T
    fca, fcb_w = fcwt[:e], fcwt[e:]
    out_dim = fcW.shape[0]

    grid = (s // block_s,)
    full = lambda arr: pl.BlockSpec(arr.shape, lambda i: (0,) * arr.ndim)
    return pl.pallas_call(
        functools.partial(_dense_kernel, hdim=h, precision=precision),
        grid=grid,
        in_specs=[
            pl.BlockSpec((block_s, e), lambda i: (i, 0)),
            full(w0f), full(w0b), full(w1f), full(w1b),
            full(b0f), full(b0b), full(b1f), full(b1b),
            full(fca), full(fcb_w), full(fcb[None, :]),
        ],
        out_specs=pl.BlockSpec((block_s, out_dim), lambda i: (i, 0)),
        out_shape=jax.ShapeDtypeStruct((s, out_dim), jnp.float32),
        interpret=interpret,
    )(xe, w0f, w0b, w1f, w1b, b0f, b0b, b1f, b1b, fca, fcb_w, fcb[None, :])


def kernel(x, x_len, emb, lstm, fcW, fcb):
    del x_len
    xe = _sc_gather(emb, x.reshape(-1).astype(jnp.int32))
    return _dense(xe, lstm, fcW, fcb)
